# baseline (speedup 1.0000x reference)
import functools

import jax
import jax.numpy as jnp
from jax import lax
from jax.experimental import pallas as pl
from jax.experimental.pallas import tpu as pltpu
from jax.experimental.pallas import tpu_sc as plsc

_TB = 4
_NC = 2
_NS = 16
_NW = _NC * _NS
_CB = 8


def _make_qz_sc(B, CAT, L):
    b_per_w = B // _NW
    mesh = plsc.VectorSubcoreMesh(core_axis_name="c", subcore_axis_name="s")

    @functools.partial(
        pl.kernel, mesh=mesh,
        out_type=jax.ShapeDtypeStruct((B * CAT * L,), jnp.float32),
        scratch_types=[
            pltpu.VMEM((CAT * 16,), jnp.float32),
            pltpu.VMEM((CAT * 16,), jnp.float32),
            pltpu.VMEM((L,), jnp.float32),
            pltpu.VMEM((_CB * L,), jnp.float32),
            pltpu.VMEM((_CB * L,), jnp.float32),
            pltpu.SemaphoreType.DMA,
            pltpu.SemaphoreType.DMA,
        ],
    )
    def qz_sc(x_hbm, w_hbm, be_hbm, out_hbm, wv, bev, xv, r0, r1, s0, s1):
        wid = lax.axis_index("s") * _NC + lax.axis_index("c")
        pltpu.sync_copy(w_hbm, wv)
        pltpu.sync_copy(be_hbm, bev)
        bufs = (r0, r1)
        sems = (s0, s1)
        pending = [None, None]
        for bi in range(b_per_w):
            b = wid * b_per_w + bi
            pltpu.sync_copy(x_hbm.at[pl.ds(b * L, L)], xv)
            for blk in range(CAT // _CB):
                p = blk % 2
                if pending[p] is not None:
                    pending[p].wait()
                    pending[p] = None
                buf = bufs[p]
                for j in range(_CB):
                    c = blk * _CB + j
                    wvec = wv[pl.ds(c * 16, 16)]
                    bvec = bev[pl.ds(c * 16, 16)]

                    def chunk(i, carry, wvec=wvec, bvec=bvec, buf=buf, j=j):
                        v = xv[pl.ds(i * 16, 16)] * wvec + bvec
                        buf[pl.ds(j * L + i * 16, 16)] = v
                        return carry

                    lax.fori_loop(0, L // 16, chunk, 0)
                row0 = (b * CAT + blk * _CB) * L
                cp = pltpu.make_async_copy(
                    buf, out_hbm.at[pl.ds(row0, _CB * L)], sems[p])
                cp.start()
                pending[p] = cp
        for p in range(2):
            if pending[p] is not None:
                pending[p].wait()

    return qz_sc


def _vae_body(inv_t_ref, x_ref, u_ref, w_ref, be_ref, wd_ref, bd_ref, px_ref):
    eps = 1e-20
    for i in range(_TB):
        qz = x_ref[i] * w_ref[:] + be_ref[:]
        g = -jnp.log(-jnp.log(u_ref[i] + eps) + eps)
        z = (qz + g) * inv_t_ref[0, 0]
        z = z - jnp.max(z, axis=0, keepdims=True)
        e = jnp.exp(z)
        y = e / jnp.sum(e, axis=0, keepdims=True)
        px = jnp.dot(wd_ref[:], y, preferred_element_type=jnp.float32)
        px_ref[i] = px + bd_ref[:]


def kernel(x, temperature, U, w_enc, b_enc, W_dec, b_dec):
    B, L = x.shape
    CAT = w_enc.shape[0]
    OUT = W_dec.shape[0]
    inv_t = (jnp.float32(1.0) / jnp.asarray(temperature, jnp.float32)).reshape(1, 1)
    w = w_enc.reshape(CAT, 1)
    be = b_enc.reshape(CAT, 1)
    bd = b_dec.reshape(OUT, 1)

    qz = _make_qz_sc(B, CAT, L)(
        x.reshape(B * L), jnp.repeat(w_enc, 16), jnp.repeat(b_enc, 16))
    qz = qz.reshape(B, CAT, L)

    px, = pl.pallas_call(
        _vae_body,
        grid=(B // _TB,),
        in_specs=[
            pl.BlockSpec(memory_space=pltpu.SMEM),
            pl.BlockSpec((_TB, 1, L), lambda b: (b, 0, 0)),
            pl.BlockSpec((_TB, CAT, L), lambda b: (b, 0, 0)),
            pl.BlockSpec((CAT, 1), lambda b: (0, 0)),
            pl.BlockSpec((CAT, 1), lambda b: (0, 0)),
            pl.BlockSpec((OUT, CAT), lambda b: (0, 0)),
            pl.BlockSpec((OUT, 1), lambda b: (0, 0)),
        ],
        out_specs=[
            pl.BlockSpec((_TB, OUT, L), lambda b: (b, 0, 0)),
        ],
        out_shape=[
            jax.ShapeDtypeStruct((B, OUT, L), jnp.float32),
        ],
        compiler_params=pltpu.CompilerParams(
            dimension_semantics=("parallel",),
        ),
    )(inv_t, x.reshape(B, 1, L), U, w, be, W_dec, bd)
    return (px, qz)


# speedup vs baseline: 1.2583x; 1.2583x over previous
import functools

import jax
import jax.numpy as jnp
from jax import lax
from jax.experimental import pallas as pl
from jax.experimental.pallas import tpu as pltpu
from jax.experimental.pallas import tpu_sc as plsc

_TB = 4
_NC = 2
_NS = 16
_NW = _NC * _NS
_CB = 8


def _make_qz_sc(B, CAT, L):
    b_per_w = B // _NW
    mesh = plsc.VectorSubcoreMesh(core_axis_name="c", subcore_axis_name="s")

    @functools.partial(
        pl.kernel, mesh=mesh,
        out_type=jax.ShapeDtypeStruct((B * CAT * L,), jnp.float32),
        scratch_types=[
            pltpu.VMEM((CAT * 16,), jnp.float32),
            pltpu.VMEM((CAT * 16,), jnp.float32),
            pltpu.VMEM((L,), jnp.float32),
            pltpu.VMEM((_CB * L,), jnp.float32),
            pltpu.VMEM((_CB * L,), jnp.float32),
            pltpu.SemaphoreType.DMA,
            pltpu.SemaphoreType.DMA,
        ],
    )
    def qz_sc(x_hbm, w_hbm, be_hbm, out_hbm, wv, bev, xv, r0, r1, s0, s1):
        wid = lax.axis_index("s") * _NC + lax.axis_index("c")
        pltpu.sync_copy(w_hbm, wv)
        pltpu.sync_copy(be_hbm, bev)
        bufs = (r0, r1)
        sems = (s0, s1)
        pending = [None, None]
        for bi in range(b_per_w):
            b = wid * b_per_w + bi
            pltpu.sync_copy(x_hbm.at[pl.ds(b * L, L)], xv)
            for blk in range(CAT // _CB):
                p = blk % 2
                if pending[p] is not None:
                    pending[p].wait()
                    pending[p] = None
                buf = bufs[p]
                for j in range(_CB):
                    c = blk * _CB + j
                    wvec = wv[pl.ds(c * 16, 16)]
                    bvec = bev[pl.ds(c * 16, 16)]

                    def chunk(i, carry, wvec=wvec, bvec=bvec, buf=buf, j=j):
                        base = i * 128
                        for k in range(8):
                            o = base + k * 16
                            v = xv[pl.ds(o, 16)] * wvec + bvec
                            buf[pl.ds(j * L + o, 16)] = v
                        return carry

                    lax.fori_loop(0, L // 128, chunk, 0)
                row0 = (b * CAT + blk * _CB) * L
                cp = pltpu.make_async_copy(
                    buf, out_hbm.at[pl.ds(row0, _CB * L)], sems[p])
                cp.start()
                pending[p] = cp
        for p in range(2):
            if pending[p] is not None:
                pending[p].wait()

    return qz_sc


def _vae_body(inv_t_ref, x_ref, u_ref, w_ref, be_ref, wd_ref, bd_ref, px_ref):
    eps = 1e-20
    for i in range(_TB):
        qz = x_ref[i] * w_ref[:] + be_ref[:]
        g = -jnp.log(-jnp.log(u_ref[i] + eps) + eps)
        z = (qz + g) * inv_t_ref[0, 0]
        z = z - jnp.max(z, axis=0, keepdims=True)
        e = jnp.exp(z)
        y = e / jnp.sum(e, axis=0, keepdims=True)
        px = jnp.dot(wd_ref[:], y, preferred_element_type=jnp.float32)
        px_ref[i] = px + bd_ref[:]


def kernel(x, temperature, U, w_enc, b_enc, W_dec, b_dec):
    B, L = x.shape
    CAT = w_enc.shape[0]
    OUT = W_dec.shape[0]
    inv_t = (jnp.float32(1.0) / jnp.asarray(temperature, jnp.float32)).reshape(1, 1)
    w = w_enc.reshape(CAT, 1)
    be = b_enc.reshape(CAT, 1)
    bd = b_dec.reshape(OUT, 1)

    qz = _make_qz_sc(B, CAT, L)(
        x.reshape(B * L), jnp.repeat(w_enc, 16), jnp.repeat(b_enc, 16))
    qz = qz.reshape(B, CAT, L)

    px, = pl.pallas_call(
        _vae_body,
        grid=(B // _TB,),
        in_specs=[
            pl.BlockSpec(memory_space=pltpu.SMEM),
            pl.BlockSpec((_TB, 1, L), lambda b: (b, 0, 0)),
            pl.BlockSpec((_TB, CAT, L), lambda b: (b, 0, 0)),
            pl.BlockSpec((CAT, 1), lambda b: (0, 0)),
            pl.BlockSpec((CAT, 1), lambda b: (0, 0)),
            pl.BlockSpec((OUT, CAT), lambda b: (0, 0)),
            pl.BlockSpec((OUT, 1), lambda b: (0, 0)),
        ],
        out_specs=[
            pl.BlockSpec((_TB, OUT, L), lambda b: (b, 0, 0)),
        ],
        out_shape=[
            jax.ShapeDtypeStruct((B, OUT, L), jnp.float32),
        ],
        compiler_params=pltpu.CompilerParams(
            dimension_semantics=("parallel",),
        ),
    )(inv_t, x.reshape(B, 1, L), U, w, be, W_dec, bd)
    return (px, qz)


# speedup vs baseline: 2.0678x; 1.6433x over previous
import jax
import jax.numpy as jnp
from jax.experimental import pallas as pl
from jax.experimental.pallas import tpu as pltpu

_TB = 4


def _vae_body(inv_t_ref, x_ref, u_ref, w_ref, be_ref, wd_ref, bd_ref,
              px_ref, qz_ref):
    eps = 1e-20
    for i in range(_TB):
        qz = x_ref[i] * w_ref[:] + be_ref[:]
        g = -jnp.log(-jnp.log(u_ref[i] + eps) + eps)
        z = (qz + g) * inv_t_ref[0, 0]
        z = z - jnp.max(z, axis=0, keepdims=True)
        e = jnp.exp(z)
        y = e / jnp.sum(e, axis=0, keepdims=True)
        px = jnp.dot(wd_ref[:], y, preferred_element_type=jnp.float32)
        px_ref[i] = px + bd_ref[:]
        qz_ref[i] = qz


def kernel(x, temperature, U, w_enc, b_enc, W_dec, b_dec):
    B, L = x.shape
    CAT = w_enc.shape[0]
    OUT = W_dec.shape[0]
    inv_t = (jnp.float32(1.0) / jnp.asarray(temperature, jnp.float32)).reshape(1, 1)
    w = w_enc.reshape(CAT, 1)
    be = b_enc.reshape(CAT, 1)
    bd = b_dec.reshape(OUT, 1)

    px, qz = pl.pallas_call(
        _vae_body,
        grid=(B // _TB,),
        in_specs=[
            pl.BlockSpec(memory_space=pltpu.SMEM),
            pl.BlockSpec((_TB, 1, L), lambda b: (b, 0, 0)),
            pl.BlockSpec((_TB, CAT, L), lambda b: (b, 0, 0)),
            pl.BlockSpec((CAT, 1), lambda b: (0, 0)),
            pl.BlockSpec((CAT, 1), lambda b: (0, 0)),
            pl.BlockSpec((OUT, CAT), lambda b: (0, 0)),
            pl.BlockSpec((OUT, 1), lambda b: (0, 0)),
        ],
        out_specs=[
            pl.BlockSpec((_TB, OUT, L), lambda b: (b, 0, 0)),
            pl.BlockSpec((_TB, CAT, L), lambda b: (b, 0, 0)),
        ],
        out_shape=[
            jax.ShapeDtypeStruct((B, OUT, L), jnp.float32),
            jax.ShapeDtypeStruct((B, CAT, L), jnp.float32),
        ],
        compiler_params=pltpu.CompilerParams(
            dimension_semantics=("parallel",),
        ),
    )(inv_t, x.reshape(B, 1, L), U, w, be, W_dec, bd)
    return (px, qz)
